# baseline (speedup 1.0000x reference)
import functools

import jax
import jax.numpy as jnp
from jax import lax
from jax.experimental import pallas as pl
from jax.experimental.pallas import tpu as pltpu
from jax.experimental.pallas import tpu_sc as plsc

B = 4096
N = 2048
XROW = 3 * N
OROW = 5 * N
L = 16
G = N // L
UNROLL = 8


def _sc_call(xf, pe0, pe1):
    info = plsc.get_sparse_core_info()
    nc, ns = info.num_cores, info.num_subcores
    nw = nc * ns
    rows_per_w = B // nw

    mesh = plsc.VectorSubcoreMesh(core_axis_name="c", subcore_axis_name="s")

    @functools.partial(
        pl.kernel,
        mesh=mesh,
        compiler_params=pltpu.CompilerParams(needs_layout_passes=False),
        out_type=[
            jax.ShapeDtypeStruct((B * OROW,), jnp.float32),
            jax.ShapeDtypeStruct((B * N,), jnp.int32),
        ],
        scratch_types=[
            pltpu.VMEM((XROW,), jnp.float32),
            pltpu.VMEM((OROW,), jnp.float32),
            pltpu.VMEM((N,), jnp.int32),
            pltpu.VMEM((N,), jnp.float32),
            pltpu.VMEM((N,), jnp.float32),
        ],
    )
    def k(x_hbm, pe0_hbm, pe1_hbm, out_hbm, mask_hbm, xin, obuf, mbuf, pe0v, pe1v):
        wid = lax.axis_index("s") * nc + lax.axis_index("c")
        pltpu.sync_copy(pe0_hbm, pe0v)
        pltpu.sync_copy(pe1_hbm, pe1v)
        iota = lax.iota(jnp.int32, L)
        gidx = iota * 3
        sidx = iota * 5
        zero = jnp.zeros((L,), jnp.float32)

        def row_body(i, carry):
            r = wid * rows_per_w + i
            pltpu.sync_copy(x_hbm.at[pl.ds(r * XROW, XROW)], xin)

            def grp(gi, c):
                for u in range(UNROLL):
                    g = gi * UNROLL + u
                    i0 = gidx + g * (3 * L)
                    s0 = sidx + g * (5 * L)
                    v0 = plsc.load_gather(xin, [i0])
                    v1 = plsc.load_gather(xin, [i0 + 1])
                    v2 = plsc.load_gather(xin, [i0 + 2])
                    p0 = pe0v[pl.ds(g * L, L)]
                    p1 = pe1v[pl.ds(g * L, L)]
                    m = v2 > 0.0
                    plsc.store_scatter(obuf, [s0], jnp.where(m, v0, zero))
                    plsc.store_scatter(obuf, [s0 + 1], jnp.where(m, v1, zero))
                    plsc.store_scatter(obuf, [s0 + 2], jnp.where(m, v2, zero))
                    plsc.store_scatter(obuf, [s0 + 3], jnp.where(m, p0, zero))
                    plsc.store_scatter(obuf, [s0 + 4], jnp.where(m, p1, zero))
                    mbuf[pl.ds(g * L, L)] = jnp.where(m, 1, 0)
                return c

            lax.fori_loop(0, G // UNROLL, grp, 0)
            pltpu.sync_copy(obuf, out_hbm.at[pl.ds(r * OROW, OROW)])
            pltpu.sync_copy(mbuf, mask_hbm.at[pl.ds(r * N, N)])
            return carry

        lax.fori_loop(0, rows_per_w, row_body, 0)

    return k(xf, pe0, pe1)


def kernel(x, pe):
    xf = x.reshape(-1)
    pe0 = jnp.asarray(pe[:, 0])
    pe1 = jnp.asarray(pe[:, 1])
    out_f, mask_i = _sc_call(xf, pe0, pe1)
    return out_f.reshape(B, N, 5), mask_i.reshape(B, N).astype(bool)


# speedup vs baseline: 13.8750x; 13.8750x over previous
import functools

import jax
import jax.numpy as jnp
from jax import lax
from jax.experimental import pallas as pl
from jax.experimental.pallas import tpu as pltpu
from jax.experimental.pallas import tpu_sc as plsc

B = 4096
N = 2048
XROW = 3 * N
L = 16
G = N // L
UNROLL = 8


def _sc_call(x, pe0, pe1):
    info = plsc.get_sparse_core_info()
    nc, ns = info.num_cores, info.num_subcores
    nw = nc * ns
    rows_per_w = B // nw

    mesh = plsc.VectorSubcoreMesh(core_axis_name="c", subcore_axis_name="s")

    @functools.partial(
        pl.kernel,
        mesh=mesh,
        compiler_params=pltpu.CompilerParams(needs_layout_passes=False),
        out_type=[
            jax.ShapeDtypeStruct((5, B, N), jnp.float32),
            jax.ShapeDtypeStruct((B, N), jnp.int32),
        ],
        scratch_types=[
            pltpu.VMEM((XROW,), jnp.float32),
            pltpu.VMEM((5 * N,), jnp.float32),
            pltpu.VMEM((N,), jnp.int32),
            pltpu.VMEM((N,), jnp.float32),
            pltpu.VMEM((N,), jnp.float32),
        ],
    )
    def k(x_hbm, pe0_hbm, pe1_hbm, out_hbm, mask_hbm, xin, obuf, mbuf, pe0v, pe1v):
        wid = lax.axis_index("s") * nc + lax.axis_index("c")
        pltpu.sync_copy(pe0_hbm, pe0v)
        pltpu.sync_copy(pe1_hbm, pe1v)
        iota = lax.iota(jnp.int32, L)
        gidx = iota * 3
        zero = jnp.zeros((L,), jnp.float32)

        def row_body(i, carry):
            r = wid * rows_per_w + i
            pltpu.sync_copy(x_hbm.at[r], xin)

            def grp(gi, c):
                for u in range(UNROLL):
                    g = gi * UNROLL + u
                    i0 = gidx + g * (3 * L)
                    v0 = plsc.load_gather(xin, [i0])
                    v1 = plsc.load_gather(xin, [i0 + 1])
                    v2 = plsc.load_gather(xin, [i0 + 2])
                    p0 = pe0v[pl.ds(g * L, L)]
                    p1 = pe1v[pl.ds(g * L, L)]
                    m = v2 > 0.0
                    obuf[pl.ds(0 * N + g * L, L)] = jnp.where(m, v0, zero)
                    obuf[pl.ds(1 * N + g * L, L)] = jnp.where(m, v1, zero)
                    obuf[pl.ds(2 * N + g * L, L)] = jnp.where(m, v2, zero)
                    obuf[pl.ds(3 * N + g * L, L)] = jnp.where(m, p0, zero)
                    obuf[pl.ds(4 * N + g * L, L)] = jnp.where(m, p1, zero)
                    mbuf[pl.ds(g * L, L)] = jnp.where(m, 1, 0)
                return c

            lax.fori_loop(0, G // UNROLL, grp, 0)
            for ch in range(5):
                pltpu.sync_copy(obuf.at[pl.ds(ch * N, N)], out_hbm.at[ch, r])
            pltpu.sync_copy(mbuf, mask_hbm.at[r])
            return carry

        lax.fori_loop(0, rows_per_w, row_body, 0)

    return k(x, pe0, pe1)


def kernel(x, pe):
    pe0 = jnp.asarray(pe[:, 0])
    pe1 = jnp.asarray(pe[:, 1])
    out_p, mask_i = _sc_call(x, pe0, pe1)
    return jnp.moveaxis(out_p, 0, -1), mask_i.astype(bool)


# speedup vs baseline: 29.3704x; 2.1168x over previous
import functools

import jax
import jax.numpy as jnp
from jax import lax
from jax.experimental import pallas as pl
from jax.experimental.pallas import tpu as pltpu
from jax.experimental.pallas import tpu_sc as plsc

B = 4096
N = 2048
XROW = 3 * N
L = 16
G = N // L
UNROLL = 8


def _sc_call(x, pe0, pe1):
    info = plsc.get_sparse_core_info()
    nc, ns = info.num_cores, info.num_subcores
    nw = nc * ns
    rows_per_w = B // nw

    mesh = plsc.VectorSubcoreMesh(core_axis_name="c", subcore_axis_name="s")

    @functools.partial(
        pl.kernel,
        mesh=mesh,
        compiler_params=pltpu.CompilerParams(needs_layout_passes=False),
        out_type=[
            jax.ShapeDtypeStruct((5, B, N), jnp.float32),
            jax.ShapeDtypeStruct((B, N), jnp.int32),
        ],
        scratch_types=[
            pltpu.VMEM((XROW,), jnp.float32),
            pltpu.VMEM((XROW,), jnp.float32),
            pltpu.VMEM((5 * N,), jnp.float32),
            pltpu.VMEM((5 * N,), jnp.float32),
            pltpu.VMEM((N,), jnp.int32),
            pltpu.VMEM((N,), jnp.int32),
            pltpu.VMEM((N,), jnp.float32),
            pltpu.VMEM((N,), jnp.float32),
            pltpu.SemaphoreType.DMA,
            pltpu.SemaphoreType.DMA,
            pltpu.SemaphoreType.DMA,
            pltpu.SemaphoreType.DMA,
        ],
    )
    def k(x_hbm, pe0_hbm, pe1_hbm, out_hbm, mask_hbm,
          xin0, xin1, ob0, ob1, mb0, mb1, pe0v, pe1v,
          sin0, sin1, sout0, sout1):
        wid = lax.axis_index("s") * nc + lax.axis_index("c")
        r0 = wid * rows_per_w
        pltpu.sync_copy(pe0_hbm, pe0v)
        pltpu.sync_copy(pe1_hbm, pe1v)
        iota = lax.iota(jnp.int32, L)
        gidx = iota * 3
        zero = jnp.zeros((L,), jnp.float32)

        xin = (xin0, xin1)
        ob = (ob0, ob1)
        mb = (mb0, mb1)
        sin = (sin0, sin1)
        sout = (sout0, sout1)

        def issue_in(r, b):
            pltpu.async_copy(x_hbm.at[r], xin[b], sin[b])

        def wait_in(b):
            pltpu.make_async_copy(x_hbm.at[0], xin[b], sin[b]).wait()

        def issue_out(r, b):
            for ch in range(5):
                pltpu.async_copy(ob[b].at[pl.ds(ch * N, N)],
                                 out_hbm.at[ch, r], sout[b])
            pltpu.async_copy(mb[b], mask_hbm.at[r], sout[b])

        def wait_out(r, b):
            for ch in range(5):
                pltpu.make_async_copy(ob[b].at[pl.ds(ch * N, N)],
                                      out_hbm.at[ch, r], sout[b]).wait()
            pltpu.make_async_copy(mb[b], mask_hbm.at[r], sout[b]).wait()

        def compute(b):
            obuf = ob[b]
            mbuf = mb[b]
            xbuf = xin[b]

            def grp(gi, c):
                for u in range(UNROLL):
                    g = gi * UNROLL + u
                    i0 = gidx + g * (3 * L)
                    v0 = plsc.load_gather(xbuf, [i0])
                    v1 = plsc.load_gather(xbuf, [i0 + 1])
                    v2 = plsc.load_gather(xbuf, [i0 + 2])
                    p0 = pe0v[pl.ds(g * L, L)]
                    p1 = pe1v[pl.ds(g * L, L)]
                    m = v2 > 0.0
                    obuf[pl.ds(0 * N + g * L, L)] = jnp.where(m, v0, zero)
                    obuf[pl.ds(1 * N + g * L, L)] = jnp.where(m, v1, zero)
                    obuf[pl.ds(2 * N + g * L, L)] = jnp.where(m, v2, zero)
                    obuf[pl.ds(3 * N + g * L, L)] = jnp.where(m, p0, zero)
                    obuf[pl.ds(4 * N + g * L, L)] = jnp.where(m, p1, zero)
                    mbuf[pl.ds(g * L, L)] = jnp.where(m, 1, 0)
                return c

            lax.fori_loop(0, G // UNROLL, grp, 0)

        issue_in(r0, 0)
        issue_in(r0 + 1, 1)

        def pair_body(j, carry):
            for b in range(2):
                i = 2 * j + b
                r = r0 + i
                wait_in(b)

                @pl.when(j > 0)
                def _():
                    wait_out(r - 2, b)

                compute(b)
                issue_out(r, b)

                @pl.when(i + 2 < rows_per_w)
                def _():
                    issue_in(r + 2, b)

            return carry

        lax.fori_loop(0, rows_per_w // 2, pair_body, 0)
        wait_out(r0 + rows_per_w - 2, 0)
        wait_out(r0 + rows_per_w - 1, 1)

    return k(x, pe0, pe1)


def kernel(x, pe):
    pe0 = jnp.asarray(pe[:, 0])
    pe1 = jnp.asarray(pe[:, 1])
    out_p, mask_i = _sc_call(x, pe0, pe1)
    return jnp.moveaxis(out_p, 0, -1), mask_i.astype(bool)


# speedup vs baseline: 31.6208x; 1.0766x over previous
import functools

import jax
import jax.numpy as jnp
from jax import lax
from jax.experimental import pallas as pl
from jax.experimental.pallas import tpu as pltpu
from jax.experimental.pallas import tpu_sc as plsc

B = 4096
N = 2048
XROW = 3 * N
L = 16
G = N // L
UNROLL = 8
NBUF = 4


def _sc_call(x, pe0, pe1):
    info = plsc.get_sparse_core_info()
    nc, ns = info.num_cores, info.num_subcores
    nw = nc * ns
    rows_per_w = B // nw

    mesh = plsc.VectorSubcoreMesh(core_axis_name="c", subcore_axis_name="s")

    scratch = (
        [pltpu.VMEM((XROW,), jnp.float32) for _ in range(NBUF)]
        + [pltpu.VMEM((5 * N,), jnp.float32) for _ in range(NBUF)]
        + [pltpu.VMEM((N,), jnp.int32) for _ in range(NBUF)]
        + [pltpu.VMEM((N,), jnp.float32), pltpu.VMEM((N,), jnp.float32)]
        + [pltpu.SemaphoreType.DMA for _ in range(2 * NBUF)]
    )

    @functools.partial(
        pl.kernel,
        mesh=mesh,
        compiler_params=pltpu.CompilerParams(needs_layout_passes=False),
        out_type=[
            jax.ShapeDtypeStruct((5, B, N), jnp.float32),
            jax.ShapeDtypeStruct((B, N), jnp.int32),
        ],
        scratch_types=scratch,
    )
    def k(x_hbm, pe0_hbm, pe1_hbm, out_hbm, mask_hbm, *bufs):
        xin = bufs[0:NBUF]
        ob = bufs[NBUF:2 * NBUF]
        mb = bufs[2 * NBUF:3 * NBUF]
        pe0v, pe1v = bufs[3 * NBUF], bufs[3 * NBUF + 1]
        sin = bufs[3 * NBUF + 2:3 * NBUF + 2 + NBUF]
        sout = bufs[3 * NBUF + 2 + NBUF:]

        wid = lax.axis_index("s") * nc + lax.axis_index("c")
        r0 = wid * rows_per_w
        pltpu.sync_copy(pe0_hbm, pe0v)
        pltpu.sync_copy(pe1_hbm, pe1v)
        iota = lax.iota(jnp.int32, L)
        gidx = iota * 3
        zero = jnp.zeros((L,), jnp.float32)

        def issue_in(r, b):
            pltpu.async_copy(x_hbm.at[r], xin[b], sin[b])

        def wait_in(b):
            pltpu.make_async_copy(x_hbm.at[0], xin[b], sin[b]).wait()

        def issue_out(r, b):
            for ch in range(5):
                pltpu.async_copy(ob[b].at[pl.ds(ch * N, N)],
                                 out_hbm.at[ch, r], sout[b])
            pltpu.async_copy(mb[b], mask_hbm.at[r], sout[b])

        def wait_out(r, b):
            for ch in range(5):
                pltpu.make_async_copy(ob[b].at[pl.ds(ch * N, N)],
                                      out_hbm.at[ch, r], sout[b]).wait()
            pltpu.make_async_copy(mb[b], mask_hbm.at[r], sout[b]).wait()

        def compute(b):
            obuf = ob[b]
            mbuf = mb[b]
            xbuf = xin[b]

            def grp(gi, c):
                for u in range(UNROLL):
                    g = gi * UNROLL + u
                    i0 = gidx + g * (3 * L)
                    v0 = plsc.load_gather(xbuf, [i0])
                    v1 = plsc.load_gather(xbuf, [i0 + 1])
                    v2 = plsc.load_gather(xbuf, [i0 + 2])
                    p0 = pe0v[pl.ds(g * L, L)]
                    p1 = pe1v[pl.ds(g * L, L)]
                    m = v2 > 0.0
                    obuf[pl.ds(0 * N + g * L, L)] = jnp.where(m, v0, zero)
                    obuf[pl.ds(1 * N + g * L, L)] = jnp.where(m, v1, zero)
                    obuf[pl.ds(2 * N + g * L, L)] = jnp.where(m, v2, zero)
                    obuf[pl.ds(3 * N + g * L, L)] = jnp.where(m, p0, zero)
                    obuf[pl.ds(4 * N + g * L, L)] = jnp.where(m, p1, zero)
                    mbuf[pl.ds(g * L, L)] = jnp.where(m, 1, 0)
                return c

            lax.fori_loop(0, G // UNROLL, grp, 0)

        for b in range(NBUF):
            issue_in(r0 + b, b)

        def ring_body(j, carry):
            for b in range(NBUF):
                i = NBUF * j + b
                r = r0 + i
                wait_in(b)

                @pl.when(j > 0)
                def _():
                    wait_out(r - NBUF, b)

                compute(b)
                issue_out(r, b)

                @pl.when(i + NBUF < rows_per_w)
                def _():
                    issue_in(r + NBUF, b)

            return carry

        lax.fori_loop(0, rows_per_w // NBUF, ring_body, 0)
        for b in range(NBUF):
            wait_out(r0 + rows_per_w - NBUF + b, b)

    return k(x, pe0, pe1)


def kernel(x, pe):
    pe0 = jnp.asarray(pe[:, 0])
    pe1 = jnp.asarray(pe[:, 1])
    out_p, mask_i = _sc_call(x, pe0, pe1)
    return jnp.moveaxis(out_p, 0, -1), mask_i.astype(bool)
